# Initial kernel scaffold; baseline (speedup 1.0000x reference)
import jax
import jax.numpy as jnp
from jax.experimental import pallas as pl


def kernel(x, edge_index, batch, target, params):
    raise NotImplementedError("write your pallas kernel here")


# speedup vs baseline: 1.3226x; 1.3226x over previous
import functools

import jax
import jax.numpy as jnp
import numpy as np
from jax import lax
from jax.experimental import pallas as pl
from jax.experimental.pallas import tpu as pltpu
from jax.experimental.pallas import tpu_sc as plsc

N = 50000
NP = 50048
E = 800000
B = 128
FEAT = 78
FEATP = 96
DIM = 32
VOCAB = 8000
EMB = 100
SEQ = 1000
OUT = 128

NC = 2
NS = 16
NW = NC * NS

RSHIFT = 11
RANGE = 1 << RSHIFT
ACC_ROWS = 51200
CHUNK_E = 2048
EPAD2 = 423 * CHUNK_E

SNBLK = 14
SPAD = NW * SNBLK * CHUNK_E // 16
EBLK = 128
POOL_ROWS = 160

TBLK = 125
TNBLK = 32

NGRID = NP // 128

_EPS = 1e-5
_INV_N = np.float32(1.0 / N)


def _sc_mesh():
    return plsc.VectorSubcoreMesh(
        core_axis_name="c", subcore_axis_name="s", num_cores=NC, num_subcores=NS)


@functools.partial(
    pl.kernel,
    out_type=jax.ShapeDtypeStruct((ACC_ROWS, DIM), jnp.float32),
    mesh=_sc_mesh(),
    compiler_params=pltpu.CompilerParams(use_tc_tiling_on_sc=False,
                                         needs_layout_passes=False),
    scratch_types=[
        pltpu.VMEM((16, EBLK), jnp.int32),
        pltpu.VMEM((16, EBLK), jnp.int32),
        pltpu.VMEM((EBLK, DIM), jnp.float32),
        pltpu.VMEM((EBLK, DIM), jnp.float32),
        pltpu.VMEM_SHARED((ACC_ROWS, DIM), jnp.float32),
        pltpu.VMEM((32,), jnp.int32),
        pltpu.VMEM((32,), jnp.int32),
        pltpu.SemaphoreType.DMA,
        pltpu.SemaphoreType.DMA,
    ],
)
def _sc_agg(values, psrc, pdst, zeros, nchunk, cstart, out,
            src_v, dst_v, rows_a, rows_b, acc, nch_s, cst_s, sem_a, sem_b):
    cid = lax.axis_index("c")
    sid = lax.axis_index("s")
    wid = cid * NS + sid

    zr = ACC_ROWS // NS // 8
    for r in range(8):
        pltpu.sync_copy(zeros, acc.at[pl.ds(sid * (ACC_ROWS // NS) + r * zr, zr)])
    pltpu.sync_copy(nchunk, nch_s)
    pltpu.sync_copy(cstart, cst_s)
    plsc.subcore_barrier()

    def _sel(ref):
        vec = jnp.where(cid == 0, ref[pl.ds(0, 16)], ref[pl.ds(16, 16)])
        lane = lax.broadcasted_iota(jnp.int32, (16,), 0)
        return lax.reduce_max(jnp.where(lane == sid, vec, 0), (0,))

    nch = _sel(nch_s)
    cst = _sel(cst_s)

    def chunk_body(c, carry):
        base = (cst + c) * 16
        pltpu.sync_copy(psrc.at[pl.ds(base, 16)], src_v)
        pltpu.sync_copy(pdst.at[pl.ds(base, 16)], dst_v)
        pltpu.async_copy(values.at[src_v.at[0]], rows_a, sem_a)

        def pair(p, carry2):
            j = p * 2
            pltpu.make_async_copy(values.at[src_v.at[j]], rows_a, sem_a).wait()
            cp_b = pltpu.async_copy(values.at[src_v.at[j + 1]], rows_b, sem_b)
            pltpu.sync_copy(rows_a, acc.at[dst_v.at[j]], add=True)
            cp_b.wait()

            @pl.when(j + 2 < 16)
            def _():
                pltpu.async_copy(values.at[src_v.at[j + 2]], rows_a, sem_a)

            pltpu.sync_copy(rows_b, acc.at[dst_v.at[j + 1]], add=True)
            return carry2

        lax.fori_loop(0, 8, pair, 0)
        return carry

    lax.fori_loop(0, nch, chunk_body, 0)
    plsc.subcore_barrier()

    row_base = wid * RANGE

    @pl.when(row_base < ACC_ROWS)
    def _():
        pltpu.sync_copy(acc.at[pl.ds(row_base, RANGE)],
                        out.at[pl.ds(row_base, RANGE)])


@functools.partial(
    pl.kernel,
    out_type=jax.ShapeDtypeStruct((NC, POOL_ROWS, DIM), jnp.float32),
    mesh=_sc_mesh(),
    compiler_params=pltpu.CompilerParams(use_tc_tiling_on_sc=False),
    scratch_types=[
        pltpu.VMEM((SNBLK, EBLK), jnp.int32),
        pltpu.VMEM((SNBLK, EBLK), jnp.int32),
        pltpu.VMEM((EBLK, DIM), jnp.float32),
        pltpu.VMEM((EBLK, DIM), jnp.float32),
        pltpu.VMEM_SHARED((POOL_ROWS, DIM), jnp.float32),
        pltpu.SemaphoreType.DMA,
        pltpu.SemaphoreType.DMA,
    ],
)
def _sc_pool(values, srcs, dsts, zeros, out,
             src_v, dst_v, rows_a, rows_b, acc, sem_a, sem_b):
    cid = lax.axis_index("c")
    sid = lax.axis_index("s")
    wid = cid * NS + sid
    rows_per = POOL_ROWS // NS

    pltpu.sync_copy(zeros, acc.at[pl.ds(sid * rows_per, rows_per)])
    plsc.subcore_barrier()

    pltpu.sync_copy(srcs.at[wid], src_v)
    pltpu.sync_copy(dsts.at[wid], dst_v)
    pltpu.async_copy(values.at[src_v.at[0]], rows_a, sem_a)

    def pair(p, carry2):
        j = p * 2
        pltpu.make_async_copy(values.at[src_v.at[j]], rows_a, sem_a).wait()
        cp_b = pltpu.async_copy(values.at[src_v.at[j + 1]], rows_b, sem_b)
        pltpu.sync_copy(rows_a, acc.at[dst_v.at[j]], add=True)
        cp_b.wait()

        @pl.when(j + 2 < SNBLK)
        def _():
            pltpu.async_copy(values.at[src_v.at[j + 2]], rows_a, sem_a)

        pltpu.sync_copy(rows_b, acc.at[dst_v.at[j + 1]], add=True)
        return carry2

    lax.fori_loop(0, SNBLK // 2, pair, 0)
    plsc.subcore_barrier()

    rows_per16 = POOL_ROWS // NS
    pltpu.sync_copy(acc.at[pl.ds(sid * rows_per16, rows_per16)],
                    out.at[cid, pl.ds(sid * rows_per16, rows_per16)])


@functools.partial(
    pl.kernel,
    out_type=jax.ShapeDtypeStruct((B * SEQ, 128), jnp.float32),
    mesh=_sc_mesh(),
    compiler_params=pltpu.CompilerParams(use_tc_tiling_on_sc=False),
    scratch_types=[
        pltpu.VMEM((TNBLK, TBLK), jnp.int32),
        pltpu.VMEM((TBLK, 128), jnp.float32),
        pltpu.VMEM((TBLK, 128), jnp.float32),
        pltpu.SemaphoreType.DMA,
        pltpu.SemaphoreType.DMA,
    ],
)
def _sc_emb_gather(table, idxs, out, idx_v, rows_a, rows_b, sem_a, sem_b):
    cid = lax.axis_index("c")
    sid = lax.axis_index("s")
    wid = cid * NS + sid
    base = wid * (TNBLK * TBLK)

    pltpu.sync_copy(idxs.at[wid], idx_v)
    pltpu.async_copy(table.at[idx_v.at[0]], rows_a, sem_a)

    def pair(p, carry):
        j = p * 2
        pltpu.make_async_copy(table.at[idx_v.at[j]], rows_a, sem_a).wait()
        cp_b = pltpu.async_copy(table.at[idx_v.at[j + 1]], rows_b, sem_b)
        pltpu.sync_copy(rows_a, out.at[pl.ds(base + j * TBLK, TBLK)])
        cp_b.wait()

        @pl.when(j + 2 < TNBLK)
        def _():
            pltpu.async_copy(table.at[idx_v.at[j + 2]], rows_a, sem_a)

        pltpu.sync_copy(rows_b, out.at[pl.ds(base + (j + 1) * TBLK, TBLK)])
        return carry

    lax.fori_loop(0, TNBLK // 2, pair, 0)


def _colsum_finalize(a):
    t = a[0:8]
    for k in range(1, 16):
        t = t + a[8 * k:8 * k + 8]
    t = t[0:4] + t[4:8]
    t = t[0:2] + t[2:4]
    return t[0:1] + t[1:2]


def _row_mask(i, x):
    rid = lax.broadcasted_iota(jnp.int32, x.shape, 0) + i * 128
    return jnp.where(rid < N, x, 0.0)


def _make_layer_body(nsl):
    def body(*args):
        h_ref = args[0]
        aggs = args[1:1 + nsl]
        w1_ref, b1_ref, w2_ref, b2_ref = args[1 + nsl:5 + nsl]
        z2_ref, st_ref, acc_ref = args[5 + nsl:]

        i = pl.program_id(0)
        if nsl > 1:
            agg = jnp.concatenate([a[...] for a in aggs], axis=1)
        else:
            agg = aggs[0][...]
        z = jnp.dot(h_ref[...] + agg, w1_ref[...],
                    preferred_element_type=jnp.float32)
        z = jnp.maximum(z + b1_ref[...], 0.0)
        z2 = jnp.dot(z, w2_ref[...], preferred_element_type=jnp.float32)
        z2 = jnp.maximum(z2 + b2_ref[...], 0.0)
        z2_ref[...] = z2

        @pl.when(i == 0)
        def _():
            acc_ref[...] = jnp.zeros_like(acc_ref)

        acc_ref[...] += _row_mask(i, z2)

        @pl.when(i == NGRID - 1)
        def _():
            mean = _colsum_finalize(acc_ref[...]) * _INV_N
            st_ref[...] = jnp.concatenate(
                [mean, jnp.zeros((7, DIM), jnp.float32)], axis=0)

    return body


def _tc_layer(h, agg_list, w1, b1, w2, b2):
    nsl = len(agg_list)
    feat = h.shape[1]
    agg_specs = [pl.BlockSpec((128, DIM), lambda i: (i, 0)) for _ in agg_list]
    return pl.pallas_call(
        _make_layer_body(nsl),
        grid=(NGRID,),
        in_specs=[pl.BlockSpec((128, feat), lambda i: (i, 0))] + agg_specs + [
            pl.BlockSpec((feat, DIM), lambda i: (0, 0)),
            pl.BlockSpec((1, DIM), lambda i: (0, 0)),
            pl.BlockSpec((DIM, DIM), lambda i: (0, 0)),
            pl.BlockSpec((1, DIM), lambda i: (0, 0)),
        ],
        out_specs=[
            pl.BlockSpec((128, DIM), lambda i: (i, 0)),
            pl.BlockSpec((8, DIM), lambda i: (0, 0)),
        ],
        out_shape=[
            jax.ShapeDtypeStruct((NP, DIM), jnp.float32),
            jax.ShapeDtypeStruct((8, DIM), jnp.float32),
        ],
        scratch_shapes=[pltpu.VMEM((128, DIM), jnp.float32)],
    )(h, *agg_list, w1, b1, w2, b2)


def _var_body(z2_ref, st1_ref, st2_ref, acc_ref):
    i = pl.program_id(0)
    d = z2_ref[...] - st1_ref[0:1, :]
    sq = d * d

    @pl.when(i == 0)
    def _():
        acc_ref[...] = jnp.zeros_like(acc_ref)

    acc_ref[...] += _row_mask(i, sq)

    @pl.when(i == NGRID - 1)
    def _():
        var = _colsum_finalize(acc_ref[...]) * _INV_N
        st2_ref[...] = jnp.concatenate(
            [var, jnp.zeros((7, DIM), jnp.float32)], axis=0)


def _tc_var(z2, st1):
    return pl.pallas_call(
        _var_body,
        grid=(NGRID,),
        in_specs=[
            pl.BlockSpec((128, DIM), lambda i: (i, 0)),
            pl.BlockSpec((8, DIM), lambda i: (0, 0)),
        ],
        out_specs=pl.BlockSpec((8, DIM), lambda i: (0, 0)),
        out_shape=jax.ShapeDtypeStruct((8, DIM), jnp.float32),
        scratch_shapes=[pltpu.VMEM((128, DIM), jnp.float32)],
    )(z2, st1)


def _affine_body(z2_ref, st1_ref, st2_ref, gamma_ref, beta_ref, h_ref):
    h_ref[...] = ((z2_ref[...] - st1_ref[0:1, :])
                  / jnp.sqrt(st2_ref[0:1, :] + _EPS)
                  * gamma_ref[...] + beta_ref[...])


def _tc_affine(z2, st1, st2, gamma, beta):
    return pl.pallas_call(
        _affine_body,
        grid=(NGRID,),
        in_specs=[
            pl.BlockSpec((128, DIM), lambda i: (i, 0)),
            pl.BlockSpec((8, DIM), lambda i: (0, 0)),
            pl.BlockSpec((8, DIM), lambda i: (0, 0)),
            pl.BlockSpec((1, DIM), lambda i: (0, 0)),
            pl.BlockSpec((1, DIM), lambda i: (0, 0)),
        ],
        out_specs=pl.BlockSpec((128, DIM), lambda i: (i, 0)),
        out_shape=jax.ShapeDtypeStruct((NP, DIM), jnp.float32),
    )(z2, st1, st2, gamma, beta)


def _conv_body(w2d_ref, emb_ref, bc_ref, o_ref):
    g = emb_ref[0]
    q = jnp.dot(w2d_ref[...], g, preferred_element_type=jnp.float32)
    q3 = q.reshape(DIM, 8, 128)
    acc = q3[:, 0, 0:93]
    for k in range(1, 8):
        acc = acc + q3[:, k, k:k + 93]
    acc = acc + jnp.transpose(bc_ref[...])
    acc = jnp.concatenate([acc, jnp.zeros((DIM, 3), jnp.float32)], axis=1)
    o_ref[0] = acc


def _tc_conv(w2d, emb, bc):
    return pl.pallas_call(
        _conv_body,
        grid=(B,),
        in_specs=[
            pl.BlockSpec((256, SEQ), lambda b: (0, 0)),
            pl.BlockSpec((1, SEQ, 128), lambda b: (b, 0, 0)),
            pl.BlockSpec((1, DIM), lambda b: (0, 0)),
        ],
        out_specs=pl.BlockSpec((1, DIM, 96), lambda b: (b, 0, 0)),
        out_shape=jax.ShapeDtypeStruct((B, DIM, 96), jnp.float32),
    )(w2d, emb, bc)


def _head_body(pool_ref, conv_ref, wxt_ref, bxt_ref, wxd_ref, bxd_ref,
               w1_ref, bm1_ref, w2_ref, bm2_ref, w3_ref, bm3_ref, o_ref):
    pooled = pool_ref[0, :B, :] + pool_ref[1, :B, :]
    drug = jnp.dot(pooled, wxd_ref[...], preferred_element_type=jnp.float32)
    drug = jnp.maximum(drug + bxd_ref[...], 0.0)
    prot = jnp.dot(conv_ref[...], wxt_ref[...],
                   preferred_element_type=jnp.float32) + bxt_ref[...]
    hid = jnp.concatenate([drug, prot], axis=1)
    hid = jnp.dot(hid, w1_ref[...], preferred_element_type=jnp.float32)
    hid = jnp.maximum(hid + bm1_ref[...], 0.0)
    hid = jnp.dot(hid, w2_ref[...], preferred_element_type=jnp.float32)
    hid = jnp.maximum(hid + bm2_ref[...], 0.0)
    o_ref[...] = jnp.dot(hid, w3_ref[...],
                         preferred_element_type=jnp.float32) + bm3_ref[...]


def _tc_head(pool, conv_flat, wxt, bxt, wxd, bxd, w1, bm1, w2, bm2, w3, bm3):
    full = lambda a: pl.BlockSpec(a.shape, lambda: tuple(0 for _ in a.shape))
    args = (pool, conv_flat, wxt, bxt, wxd, bxd, w1, bm1, w2, bm2, w3, bm3)
    return pl.pallas_call(
        _head_body,
        in_specs=[full(a) for a in args],
        out_specs=pl.BlockSpec((B, 1), lambda: (0, 0)),
        out_shape=jax.ShapeDtypeStruct((B, 1), jnp.float32),
    )(*args)


def kernel(x, edge_index, batch, target, params):
    f32 = jnp.float32
    i32 = jnp.int32

    src0, dst0 = edge_index[0], edge_index[1]
    bucket = dst0 >> RSHIFT
    perm = jnp.argsort(bucket)
    srcs_s = src0[perm]
    dsts_s = dst0[perm]
    bsort = bucket[perm]
    cnt = jnp.bincount(bucket, length=32).astype(i32)
    plen = ((cnt + (CHUNK_E - 1)) // CHUNK_E) * CHUNK_E
    poff = jnp.concatenate([jnp.zeros((1,), i32),
                            jnp.cumsum(plen)[:-1].astype(i32)])
    off = jnp.concatenate([jnp.zeros((1,), i32),
                           jnp.cumsum(cnt)[:-1].astype(i32)])
    rank = jnp.arange(E, dtype=i32) - off[bsort]
    pos = poff[bsort] + rank
    ar = jnp.arange(EPAD2, dtype=i32)
    psrc = (ar % N).at[pos].set(srcs_s).reshape(-1, EBLK)
    pdst = (N + (ar % 1024)).at[pos].set(dsts_s).reshape(-1, EBLK)
    nchunk = (plen // CHUNK_E).astype(i32)
    cstart = (poff // CHUNK_E).astype(i32)

    seg_src = jnp.concatenate(
        [jnp.arange(N, dtype=i32), jnp.zeros((SPAD - N,), i32)]
    ).reshape(NW, SNBLK, EBLK)
    seg_dst = jnp.concatenate(
        [batch, jnp.full((SPAD - N,), B, i32)]
    ).reshape(NW, SNBLK, EBLK)

    zeros_agg = jnp.zeros((ACC_ROWS // NS // 8, DIM), f32)
    zeros_pool = jnp.zeros((POOL_ROWS // NS, DIM), f32)

    table = jnp.pad(params["emb"], ((0, 0), (0, 128 - EMB)))
    tgt = target.reshape(NW, TNBLK, TBLK).astype(i32)
    w2d = params["conv_xt_W"].transpose(0, 2, 1).reshape(256, SEQ)
    wxt = jnp.pad(params["fc_xt_W"].reshape(DIM, 93, OUT),
                  ((0, 0), (0, 3), (0, 0))).reshape(DIM * 96, OUT)

    xp = jnp.pad(x, ((0, NP - N), (0, FEATP - FEAT)))
    w1 = jnp.pad(params["g0_W1"], ((0, FEATP - FEAT), (0, 0)))

    row = lambda v: v.reshape(1, -1)

    def _chain(a, token):
        a, _ = lax.optimization_barrier((a, token))
        return a

    emb = _sc_emb_gather(table, tgt).reshape(B, SEQ, 128)
    conv = _tc_conv(w2d, emb, row(params["conv_xt_b"]))
    token = emb

    h = xp
    for i in range(5):
        if i == 0:
            slices = [lax.slice(h, (0, 32 * j), (NP, 32 * j + 32))
                      for j in range(3)]
        else:
            slices = [h]
        aggs = []
        for v in slices:
            a = _sc_agg(_chain(v, token), psrc, pdst, zeros_agg,
                        nchunk, cstart)
            aggs.append(a)
            token = a
        z2, st1 = _tc_layer(h, aggs, w1, row(params[f"g{i}_b1"]),
                            params[f"g{i}_W2"], row(params[f"g{i}_b2"]))
        st2 = _tc_var(z2, st1)
        h = _tc_affine(z2, st1, st2, row(params[f"g{i}_gamma"]),
                       row(params[f"g{i}_beta"]))
        if i < 4:
            w1 = params[f"g{i+1}_W1"]

    pool = _sc_pool(_chain(h, token), seg_src, seg_dst, zeros_pool)

    out = _tc_head(
        pool, conv.reshape(B, DIM * 96), wxt, row(params["fc_xt_b"]),
        params["fc_xd_W"], row(params["fc_xd_b"]),
        params["m_W1"], row(params["m_b1"]),
        params["m_W2"], row(params["m_b2"]),
        params["m_W3"], row(params["m_b3"]),
    )
    return out
